# Initial kernel scaffold; baseline (speedup 1.0000x reference)
import functools

import jax
import jax.numpy as jnp
from jax.experimental import pallas as pl
from jax.experimental.pallas import tpu as pltpu

E = 8
TOP_K = 2
D = 1024
DFF = 1024
SH_DFF = 2048
T = 2048

BT = 256


def _router_body(x_ref, gwt_ref, comb_ref, fi_ref, pi_ref):
    x = x_ref[...]
    logits = jax.lax.dot_general(
        x, gwt_ref[...], (((1,), (0,)), ((), ())),
        preferred_element_type=jnp.float32,
        precision=jax.lax.Precision.HIGHEST)
    m = jnp.max(logits, axis=-1, keepdims=True)
    p = jnp.exp(logits - m)
    scores = p / jnp.sum(p, axis=-1, keepdims=True)

    lane = jax.lax.broadcasted_iota(jnp.int32, (T, E), 1)
    s1 = jnp.max(scores, axis=-1, keepdims=True)
    i1 = jnp.min(jnp.where(scores == s1, lane, E), axis=-1, keepdims=True)
    mask1 = lane == i1
    rest = jnp.where(mask1, -jnp.inf, scores)
    s2 = jnp.max(rest, axis=-1, keepdims=True)
    i2 = jnp.min(jnp.where(rest == s2, lane, E), axis=-1, keepdims=True)
    mask2 = lane == i2

    comb_ref[...] = jnp.where(mask1, s1, 0.0) + jnp.where(mask2, s2, 0.0)
    counts = jnp.sum(mask1.astype(jnp.float32) + mask2.astype(jnp.float32),
                     axis=0, keepdims=True)
    fi_ref[...] = counts * (float(E) / float(T * TOP_K))
    pi_ref[...] = jnp.mean(scores, axis=0, keepdims=True)


def _expert_body(x_ref, gw_ref, uw_ref, dw_ref, combt_ref, y_ref):
    e = pl.program_id(1)

    @pl.when(e == 0)
    def _():
        y_ref[...] = jnp.zeros_like(y_ref)

    x = x_ref[...]
    dims = (((1,), (1,)), ((), ()))
    g = jax.lax.dot_general(x, gw_ref[0], dims,
                            preferred_element_type=jnp.float32,
                            precision=jax.lax.Precision.HIGHEST)
    u = jax.lax.dot_general(x, uw_ref[0], dims,
                            preferred_element_type=jnp.float32,
                            precision=jax.lax.Precision.HIGHEST)
    act = g * jax.nn.sigmoid(g) * u
    o = jax.lax.dot_general(act, dw_ref[0], dims,
                            preferred_element_type=jnp.float32,
                            precision=jax.lax.Precision.HIGHEST)
    w = combt_ref[0, :]
    y_ref[...] += w[:, None] * o


def _shared_body(x_ref, ymoe_ref, sg_ref, su_ref, sd_ref, y_ref):
    x = x_ref[...]
    dims = (((1,), (1,)), ((), ()))
    g = jax.lax.dot_general(x, sg_ref[...], dims,
                            preferred_element_type=jnp.float32,
                            precision=jax.lax.Precision.HIGHEST)
    u = jax.lax.dot_general(x, su_ref[...], dims,
                            preferred_element_type=jnp.float32,
                            precision=jax.lax.Precision.HIGHEST)
    act = g * jax.nn.sigmoid(g) * u
    o = jax.lax.dot_general(act, sd_ref[...], dims,
                            preferred_element_type=jnp.float32,
                            precision=jax.lax.Precision.HIGHEST)
    y_ref[...] = ymoe_ref[...] + o


def kernel(hidden_states, gate_w, expert_gate, expert_up, expert_down,
           shared_gate, shared_up, shared_down):
    b, s, h = hidden_states.shape
    x = hidden_states.reshape(-1, h)

    comb, fi, pi = pl.pallas_call(
        _router_body,
        out_shape=[
            jax.ShapeDtypeStruct((T, E), jnp.float32),
            jax.ShapeDtypeStruct((1, E), jnp.float32),
            jax.ShapeDtypeStruct((1, E), jnp.float32),
        ],
    )(x, gate_w.T)

    comb_t = comb.T

    y_moe = pl.pallas_call(
        _expert_body,
        grid=(T // BT, E),
        in_specs=[
            pl.BlockSpec((BT, D), lambda t, e: (t, 0)),
            pl.BlockSpec((1, DFF, D), lambda t, e: (e, 0, 0)),
            pl.BlockSpec((1, DFF, D), lambda t, e: (e, 0, 0)),
            pl.BlockSpec((1, D, DFF), lambda t, e: (e, 0, 0)),
            pl.BlockSpec((1, BT), lambda t, e: (e, t)),
        ],
        out_specs=pl.BlockSpec((BT, D), lambda t, e: (t, 0)),
        out_shape=jax.ShapeDtypeStruct((T, D), jnp.float32),
        compiler_params=pltpu.CompilerParams(
            dimension_semantics=("parallel", "arbitrary")),
    )(x, expert_gate, expert_up, expert_down, comb_t)

    y = pl.pallas_call(
        _shared_body,
        grid=(T // BT,),
        in_specs=[
            pl.BlockSpec((BT, D), lambda t: (t, 0)),
            pl.BlockSpec((BT, D), lambda t: (t, 0)),
            pl.BlockSpec((SH_DFF, D), lambda t: (0, 0)),
            pl.BlockSpec((SH_DFF, D), lambda t: (0, 0)),
            pl.BlockSpec((D, SH_DFF), lambda t: (0, 0)),
        ],
        out_specs=pl.BlockSpec((BT, D), lambda t: (t, 0)),
        out_shape=jax.ShapeDtypeStruct((T, D), jnp.float32),
        compiler_params=pltpu.CompilerParams(
            dimension_semantics=("parallel",)),
    )(x, y_moe, shared_gate, shared_up, shared_down)

    return (y.reshape(b, s, h), fi.reshape(E), pi.reshape(E))


# speedup vs baseline: 1.0002x; 1.0002x over previous
import functools

import jax
import jax.numpy as jnp
from jax.experimental import pallas as pl
from jax.experimental.pallas import tpu as pltpu

E = 8
TOP_K = 2
D = 1024
DFF = 1024
SH_DFF = 2048
T = 2048

BT = 256


def _router_body(x_ref, gwt_ref, comb_ref, fi_ref, pi_ref):
    x = x_ref[...]
    logits = jax.lax.dot_general(
        x, gwt_ref[...], (((1,), (0,)), ((), ())),
        preferred_element_type=jnp.float32,
        precision=jax.lax.Precision.DEFAULT)
    m = jnp.max(logits, axis=-1, keepdims=True)
    p = jnp.exp(logits - m)
    scores = p / jnp.sum(p, axis=-1, keepdims=True)

    lane = jax.lax.broadcasted_iota(jnp.int32, (T, E), 1)
    s1 = jnp.max(scores, axis=-1, keepdims=True)
    i1 = jnp.min(jnp.where(scores == s1, lane, E), axis=-1, keepdims=True)
    mask1 = lane == i1
    rest = jnp.where(mask1, -jnp.inf, scores)
    s2 = jnp.max(rest, axis=-1, keepdims=True)
    i2 = jnp.min(jnp.where(rest == s2, lane, E), axis=-1, keepdims=True)
    mask2 = lane == i2

    comb_ref[...] = jnp.where(mask1, s1, 0.0) + jnp.where(mask2, s2, 0.0)
    counts = jnp.sum(mask1.astype(jnp.float32) + mask2.astype(jnp.float32),
                     axis=0, keepdims=True)
    fi_ref[...] = counts * (float(E) / float(T * TOP_K))
    pi_ref[...] = jnp.mean(scores, axis=0, keepdims=True)


def _expert_body(x_ref, gw_ref, uw_ref, dw_ref, combt_ref, y_ref):
    e = pl.program_id(1)

    @pl.when(e == 0)
    def _():
        y_ref[...] = jnp.zeros_like(y_ref)

    x = x_ref[...]
    dims = (((1,), (1,)), ((), ()))
    g = jax.lax.dot_general(x, gw_ref[0], dims,
                            preferred_element_type=jnp.float32,
                            precision=jax.lax.Precision.DEFAULT)
    u = jax.lax.dot_general(x, uw_ref[0], dims,
                            preferred_element_type=jnp.float32,
                            precision=jax.lax.Precision.DEFAULT)
    act = g * jax.nn.sigmoid(g) * u
    o = jax.lax.dot_general(act, dw_ref[0], dims,
                            preferred_element_type=jnp.float32,
                            precision=jax.lax.Precision.DEFAULT)
    w = combt_ref[0, 0, :]
    y_ref[...] += w[:, None] * o


def _shared_body(x_ref, ymoe_ref, sg_ref, su_ref, sd_ref, y_ref):
    x = x_ref[...]
    dims = (((1,), (1,)), ((), ()))
    g = jax.lax.dot_general(x, sg_ref[...], dims,
                            preferred_element_type=jnp.float32,
                            precision=jax.lax.Precision.DEFAULT)
    u = jax.lax.dot_general(x, su_ref[...], dims,
                            preferred_element_type=jnp.float32,
                            precision=jax.lax.Precision.DEFAULT)
    act = g * jax.nn.sigmoid(g) * u
    o = jax.lax.dot_general(act, sd_ref[...], dims,
                            preferred_element_type=jnp.float32,
                            precision=jax.lax.Precision.DEFAULT)
    y_ref[...] = ymoe_ref[...] + o


def kernel(hidden_states, gate_w, expert_gate, expert_up, expert_down,
           shared_gate, shared_up, shared_down):
    b, s, h = hidden_states.shape
    x = hidden_states.reshape(-1, h)

    comb, fi, pi = pl.pallas_call(
        _router_body,
        out_shape=[
            jax.ShapeDtypeStruct((T, E), jnp.float32),
            jax.ShapeDtypeStruct((1, E), jnp.float32),
            jax.ShapeDtypeStruct((1, E), jnp.float32),
        ],
    )(x, gate_w.T)

    comb_t = comb.T.reshape(E, 1, T)

    y_moe = pl.pallas_call(
        _expert_body,
        grid=(T // BT, E),
        in_specs=[
            pl.BlockSpec((BT, D), lambda t, e: (t, 0)),
            pl.BlockSpec((1, DFF, D), lambda t, e: (e, 0, 0)),
            pl.BlockSpec((1, DFF, D), lambda t, e: (e, 0, 0)),
            pl.BlockSpec((1, D, DFF), lambda t, e: (e, 0, 0)),
            pl.BlockSpec((1, 1, BT), lambda t, e: (e, 0, t)),
        ],
        out_specs=pl.BlockSpec((BT, D), lambda t, e: (t, 0)),
        out_shape=jax.ShapeDtypeStruct((T, D), jnp.float32),
        compiler_params=pltpu.CompilerParams(
            dimension_semantics=("parallel", "arbitrary")),
    )(x, expert_gate, expert_up, expert_down, comb_t)

    y = pl.pallas_call(
        _shared_body,
        grid=(T // BT,),
        in_specs=[
            pl.BlockSpec((BT, D), lambda t: (t, 0)),
            pl.BlockSpec((BT, D), lambda t: (t, 0)),
            pl.BlockSpec((SH_DFF, D), lambda t: (0, 0)),
            pl.BlockSpec((SH_DFF, D), lambda t: (0, 0)),
            pl.BlockSpec((D, SH_DFF), lambda t: (0, 0)),
        ],
        out_specs=pl.BlockSpec((BT, D), lambda t: (t, 0)),
        out_shape=jax.ShapeDtypeStruct((T, D), jnp.float32),
        compiler_params=pltpu.CompilerParams(
            dimension_semantics=("parallel",)),
    )(x, y_moe, shared_gate, shared_up, shared_down)

    return (y.reshape(b, s, h), fi.reshape(E), pi.reshape(E))


# speedup vs baseline: 1.7147x; 1.7143x over previous
import functools

import jax
import jax.numpy as jnp
from jax import lax
from jax.experimental import pallas as pl
from jax.experimental.pallas import tpu as pltpu
from jax.experimental.pallas import tpu_sc as plsc

E = 8
TOP_K = 2
D = 1024
DFF = 1024
SH_DFF = 2048
T = 2048
A = T * TOP_K

TILE = 256
NT = A // TILE + E
NPAD = NT * TILE

BT = 256

_DEF = jax.lax.Precision.DEFAULT
_F32 = jnp.float32


def _excl_cumsum_rows(o, cb=256):
    n = o.shape[0]
    ii = lax.broadcasted_iota(jnp.int32, (cb, cb), 0)
    jj = lax.broadcasted_iota(jnp.int32, (cb, cb), 1)
    slt = (jj < ii).astype(_F32)
    out = []
    carry = jnp.zeros((1, o.shape[1]), _F32)
    for b in range(n // cb):
        ob = o[b * cb:(b + 1) * cb]
        out.append(jax.lax.dot_general(slt, ob, (((1,), (0,)), ((), ())),
                                       preferred_element_type=_F32,
                                       precision=_DEF) + carry)
        carry = carry + jnp.sum(ob, axis=0, keepdims=True)
    return jnp.concatenate(out, axis=0)


def _router_body(x_ref, gwt_ref, w_ref, pos_ref, te_ref, tv_ref,
                 fi_ref, pi_ref):
    x = x_ref[...]
    logits = jax.lax.dot_general(
        x, gwt_ref[...], (((1,), (0,)), ((), ())),
        preferred_element_type=_F32, precision=_DEF)
    m = jnp.max(logits, axis=-1, keepdims=True)
    p = jnp.exp(logits - m)
    scores = p / jnp.sum(p, axis=-1, keepdims=True)

    lane = lax.broadcasted_iota(jnp.int32, (T, E), 1)
    s1 = jnp.max(scores, axis=-1, keepdims=True)
    i1 = jnp.min(jnp.where(scores == s1, lane, E), axis=-1, keepdims=True)
    mask1 = lane == i1
    rest = jnp.where(mask1, -jnp.inf, scores)
    s2 = jnp.max(rest, axis=-1, keepdims=True)
    i2 = jnp.min(jnp.where(rest == s2, lane, E), axis=-1, keepdims=True)
    mask2 = lane == i2

    o1 = mask1.astype(_F32)
    o2 = mask2.astype(_F32)
    c1 = jnp.sum(o1, axis=0, keepdims=True)
    counts = c1 + jnp.sum(o2, axis=0, keepdims=True)

    fi_ref[...] = counts * (float(E) / float(A))
    pi_ref[...] = jnp.mean(scores, axis=0, keepdims=True)
    w_ref[...] = jnp.concatenate([s1, s2], axis=1)

    ptiles = jnp.floor((counts + (TILE - 1)) * (1.0 / TILE))
    ei = lax.broadcasted_iota(jnp.int32, (E, E), 0)
    ej = lax.broadcasted_iota(jnp.int32, (E, E), 1)
    sltE = (ei < ej).astype(_F32)
    tile_start = jax.lax.dot_general(ptiles, sltE, (((1,), (0,)), ((), ())),
                                     preferred_element_type=_F32,
                                     precision=_DEF)
    alignoff = tile_start * float(TILE)
    total_tiles = jnp.sum(ptiles, axis=1, keepdims=True)

    rank1 = _excl_cumsum_rows(o1)
    rank2 = _excl_cumsum_rows(o2) + c1
    pos1 = jnp.sum(o1 * (alignoff + rank1), axis=1, keepdims=True)
    pos2 = jnp.sum(o2 * (alignoff + rank2), axis=1, keepdims=True)
    pos_ref[...] = jnp.concatenate([pos1, pos2], axis=1).astype(jnp.int32)

    ti = lax.broadcasted_iota(jnp.int32, (NT, E), 0).astype(_F32)
    ts_b = jnp.broadcast_to(tile_start, (NT, E))
    te = jnp.sum((ti >= ts_b).astype(_F32), axis=1, keepdims=True) - 1.0
    te_ref[...] = jnp.clip(te, 0.0, float(E - 1)).astype(jnp.int32)
    tvi = lax.broadcasted_iota(jnp.int32, (NT, 1), 0).astype(_F32)
    tv_ref[...] = (tvi < total_tiles).astype(jnp.int32)


def _grouped_body(te_ref, tv_ref, xg_ref, gw_ref, uw_ref, dw_ref, out_ref):
    i = pl.program_id(0)

    @pl.when(tv_ref[i] == 1)
    def _():
        xg = xg_ref[...]
        dims = (((1,), (1,)), ((), ()))
        g = jax.lax.dot_general(xg, gw_ref[0], dims,
                                preferred_element_type=_F32, precision=_DEF)
        u = jax.lax.dot_general(xg, uw_ref[0], dims,
                                preferred_element_type=_F32, precision=_DEF)
        act = g * jax.nn.sigmoid(g) * u
        out_ref[...] = jax.lax.dot_general(act, dw_ref[0], dims,
                                           preferred_element_type=_F32,
                                           precision=_DEF)


def _shared_body(x_ref, sg_ref, su_ref, sd_ref, y_ref):
    x = x_ref[...]
    dims = (((1,), (1,)), ((), ()))
    g = jax.lax.dot_general(x, sg_ref[...], dims,
                            preferred_element_type=_F32, precision=_DEF)
    u = jax.lax.dot_general(x, su_ref[...], dims,
                            preferred_element_type=_F32, precision=_DEF)
    act = g * jax.nn.sigmoid(g) * u
    y_ref[...] = jax.lax.dot_general(act, sd_ref[...], dims,
                                     preferred_element_type=_F32,
                                     precision=_DEF)


def _combine_body(eg1_ref, eg2_ref, w_ref, ysh_ref, y_ref):
    w1 = w_ref[:, 0]
    w2 = w_ref[:, 1]
    y_ref[...] = (w1[:, None] * eg1_ref[0] + w2[:, None] * eg2_ref[0]
                  + ysh_ref[...])


def _sc_mesh():
    return plsc.VectorSubcoreMesh(core_axis_name="c", subcore_axis_name="s")


_NW = 32
_JPW = A // _NW
_CH = 64


def _sc_scatter(x, pos_flat):
    @functools.partial(
        pl.kernel, mesh=_sc_mesh(),
        out_type=jax.ShapeDtypeStruct((NPAD, D), _F32),
        scratch_types=[pltpu.VMEM((_CH,), jnp.int32),
                       pltpu.VMEM((_CH, D), _F32)],
    )
    def k(x_hbm, pos_hbm, xg_hbm, idx_v, rows_v):
        wid = lax.axis_index("s") * 2 + lax.axis_index("c")
        base = wid * _JPW

        @pl.loop(0, _JPW, step=_CH)
        def _(c):
            jb = base + c
            pltpu.sync_copy(pos_hbm.at[pl.ds(jb, _CH)], idx_v)
            pltpu.sync_copy(x_hbm.at[pl.ds(jb % T, _CH)], rows_v)
            pltpu.sync_copy(rows_v, xg_hbm.at[idx_v])

    return k(x, pos_flat)


def _sc_gather(eout, pos_flat):
    @functools.partial(
        pl.kernel, mesh=_sc_mesh(),
        out_type=jax.ShapeDtypeStruct((A, D), _F32),
        scratch_types=[pltpu.VMEM((_CH,), jnp.int32),
                       pltpu.VMEM((_CH, D), _F32)],
    )
    def k(eout_hbm, pos_hbm, eg_hbm, idx_v, rows_v):
        wid = lax.axis_index("s") * 2 + lax.axis_index("c")
        base = wid * _JPW

        @pl.loop(0, _JPW, step=_CH)
        def _(c):
            jb = base + c
            pltpu.sync_copy(pos_hbm.at[pl.ds(jb, _CH)], idx_v)
            pltpu.sync_copy(eout_hbm.at[idx_v], rows_v)
            pltpu.sync_copy(rows_v, eg_hbm.at[pl.ds(jb, _CH)])

    return k(eout, pos_flat)


def kernel(hidden_states, gate_w, expert_gate, expert_up, expert_down,
           shared_gate, shared_up, shared_down):
    b, s, h = hidden_states.shape
    x = hidden_states.reshape(-1, h)

    w2, pos_tk, te, tv, fi, pi = pl.pallas_call(
        _router_body,
        out_shape=[
            jax.ShapeDtypeStruct((T, TOP_K), _F32),
            jax.ShapeDtypeStruct((T, TOP_K), jnp.int32),
            jax.ShapeDtypeStruct((NT, 1), jnp.int32),
            jax.ShapeDtypeStruct((NT, 1), jnp.int32),
            jax.ShapeDtypeStruct((1, E), _F32),
            jax.ShapeDtypeStruct((1, E), _F32),
        ],
    )(x, gate_w.T)

    pos_flat = pos_tk.T.reshape(A)
    te_s = te.reshape(NT)
    tv_s = tv.reshape(NT)

    xg = _sc_scatter(x, pos_flat)

    eout = pl.pallas_call(
        _grouped_body,
        grid_spec=pltpu.PrefetchScalarGridSpec(
            num_scalar_prefetch=2,
            grid=(NT,),
            in_specs=[
                pl.BlockSpec((TILE, D), lambda i, te, tv: (i, 0)),
                pl.BlockSpec((1, DFF, D), lambda i, te, tv: (te[i], 0, 0)),
                pl.BlockSpec((1, DFF, D), lambda i, te, tv: (te[i], 0, 0)),
                pl.BlockSpec((1, D, DFF), lambda i, te, tv: (te[i], 0, 0)),
            ],
            out_specs=pl.BlockSpec((TILE, D), lambda i, te, tv: (i, 0)),
        ),
        out_shape=jax.ShapeDtypeStruct((NPAD, D), _F32),
        compiler_params=pltpu.CompilerParams(
            dimension_semantics=("arbitrary",)),
    )(te_s, tv_s, xg, expert_gate, expert_up, expert_down)

    eg = _sc_gather(eout, pos_flat).reshape(TOP_K, T, D)

    y_sh = pl.pallas_call(
        _shared_body,
        grid=(T // BT,),
        in_specs=[
            pl.BlockSpec((BT, D), lambda t: (t, 0)),
            pl.BlockSpec((SH_DFF, D), lambda t: (0, 0)),
            pl.BlockSpec((SH_DFF, D), lambda t: (0, 0)),
            pl.BlockSpec((D, SH_DFF), lambda t: (0, 0)),
        ],
        out_specs=pl.BlockSpec((BT, D), lambda t: (t, 0)),
        out_shape=jax.ShapeDtypeStruct((T, D), _F32),
        compiler_params=pltpu.CompilerParams(
            dimension_semantics=("parallel",)),
    )(x, shared_gate, shared_up, shared_down)

    y = pl.pallas_call(
        _combine_body,
        grid=(T // BT,),
        in_specs=[
            pl.BlockSpec((1, BT, D), lambda t: (0, t, 0)),
            pl.BlockSpec((1, BT, D), lambda t: (1, t, 0)),
            pl.BlockSpec((BT, TOP_K), lambda t: (t, 0)),
            pl.BlockSpec((BT, D), lambda t: (t, 0)),
        ],
        out_specs=pl.BlockSpec((BT, D), lambda t: (t, 0)),
        out_shape=jax.ShapeDtypeStruct((T, D), _F32),
        compiler_params=pltpu.CompilerParams(
            dimension_semantics=("parallel",)),
    )(eg, eg, w2, y_sh)

    return (y.reshape(b, s, h), fi.reshape(E), pi.reshape(E))
